# initial kernel scaffold (unmeasured)
import functools

import jax
import jax.numpy as jnp
from jax import lax
from jax.experimental import pallas as pl
from jax.experimental.pallas import tpu as pltpu

N_DEV = 8
N_HOPS = N_DEV - 1


def _allgather_kv(K, V):
    b, s, h, d = K.shape

    def body(k_ref, v_ref, kg_ref, vg_ref,
             send_k, recv_k, send_v, recv_v):
        my = lax.axis_index("i")
        left = lax.rem(my - 1 + N_DEV, N_DEV)
        right = lax.rem(my + 1, N_DEV)

        barrier_sem = pltpu.get_barrier_semaphore()
        for nbr in (left, right):
            pl.semaphore_signal(
                barrier_sem, inc=1,
                device_id=(nbr,), device_id_type=pltpu.DeviceIdType.MESH,
            )
        pl.semaphore_wait(barrier_sem, 2)

        kg_ref[my] = k_ref[...]
        vg_ref[my] = v_ref[...]

        for t in range(N_HOPS):
            c_send = lax.rem(my - t + N_DEV, N_DEV)
            rk = pltpu.make_async_remote_copy(
                src_ref=kg_ref.at[c_send],
                dst_ref=kg_ref.at[c_send],
                send_sem=send_k.at[t],
                recv_sem=recv_k.at[t],
                device_id=(right,),
                device_id_type=pltpu.DeviceIdType.MESH,
            )
            rv = pltpu.make_async_remote_copy(
                src_ref=vg_ref.at[c_send],
                dst_ref=vg_ref.at[c_send],
                send_sem=send_v.at[t],
                recv_sem=recv_v.at[t],
                device_id=(right,),
                device_id_type=pltpu.DeviceIdType.MESH,
            )
            rk.start()
            rv.start()
            rk.wait()
            rv.wait()

        @functools.partial(
            pl.run_scoped, exit_sem=pltpu.SemaphoreType.REGULAR
        )
        def _(exit_sem):
            for nbr in (left, right):
                pl.semaphore_signal(
                    exit_sem, inc=1,
                    device_id=(nbr,), device_id_type=pltpu.DeviceIdType.MESH,
                )
            pl.semaphore_wait(exit_sem, 2)

    kg, vg = pl.pallas_call(
        body,
        out_shape=(
            jax.ShapeDtypeStruct((N_DEV, b, s, h, d), K.dtype),
            jax.ShapeDtypeStruct((N_DEV, b, s, h, d), V.dtype),
        ),
        in_specs=[
            pl.BlockSpec(memory_space=pltpu.VMEM),
            pl.BlockSpec(memory_space=pltpu.VMEM),
        ],
        out_specs=(
            pl.BlockSpec(memory_space=pltpu.VMEM),
            pl.BlockSpec(memory_space=pltpu.VMEM),
        ),
        scratch_shapes=[
            pltpu.SemaphoreType.DMA((N_HOPS,)),
            pltpu.SemaphoreType.DMA((N_HOPS,)),
            pltpu.SemaphoreType.DMA((N_HOPS,)),
            pltpu.SemaphoreType.DMA((N_HOPS,)),
        ],
        compiler_params=pltpu.CompilerParams(collective_id=0),
    )(K, V)
    return kg, vg


def kernel(Q, K, V):
    b, s, h, d = Q.shape
    kg, vg = _allgather_kv(K, V)
    Kf = kg.transpose(1, 0, 2, 3, 4).reshape(b, N_DEV * s, h, d)
    Vf = vg.transpose(1, 0, 2, 3, 4).reshape(b, N_DEV * s, h, d)

    scale = d ** -0.5
    S = jnp.einsum("bqhd,bkhd->bhqk", Q, Kf) * scale
    m = S.max(-1, keepdims=True)
    P = jnp.exp(S - m)
    P = P / P.sum(-1, keepdims=True)
    return jnp.einsum("bhqk,bkhd->bqhd", P, Vf).astype(Q.dtype)


# baseline (device time: 896730 ns/iter reference)
import functools

import jax
import jax.numpy as jnp
from jax import lax
from jax.experimental import pallas as pl
from jax.experimental.pallas import tpu as pltpu

N_DEV = 8
N_HOPS = N_DEV - 1


def _allgather_kv(K2, V2):
    b, s, hd = K2.shape

    def body(k_ref, v_ref, kg_ref, vg_ref,
             copy_sems, send_k, recv_k, send_v, recv_v):
        my = lax.axis_index("i")
        left = lax.rem(my - 1 + N_DEV, N_DEV)
        right = lax.rem(my + 1, N_DEV)

        barrier_sem = pltpu.get_barrier_semaphore()
        for nbr in (left, right):
            pl.semaphore_signal(
                barrier_sem, inc=1,
                device_id=(nbr,), device_id_type=pltpu.DeviceIdType.MESH,
            )
        pl.semaphore_wait(barrier_sem, 2)

        ck = pltpu.make_async_copy(k_ref, kg_ref.at[my], copy_sems.at[0])
        cv = pltpu.make_async_copy(v_ref, vg_ref.at[my], copy_sems.at[1])
        ck.start()
        cv.start()
        ck.wait()
        cv.wait()

        for t in range(N_HOPS):
            c_send = lax.rem(my - t + N_DEV, N_DEV)
            rk = pltpu.make_async_remote_copy(
                src_ref=kg_ref.at[c_send],
                dst_ref=kg_ref.at[c_send],
                send_sem=send_k.at[t],
                recv_sem=recv_k.at[t],
                device_id=(right,),
                device_id_type=pltpu.DeviceIdType.MESH,
            )
            rv = pltpu.make_async_remote_copy(
                src_ref=vg_ref.at[c_send],
                dst_ref=vg_ref.at[c_send],
                send_sem=send_v.at[t],
                recv_sem=recv_v.at[t],
                device_id=(right,),
                device_id_type=pltpu.DeviceIdType.MESH,
            )
            rk.start()
            rv.start()
            rk.wait()
            rv.wait()

        @functools.partial(
            pl.run_scoped, exit_sem=pltpu.SemaphoreType.REGULAR
        )
        def _(exit_sem):
            for nbr in (left, right):
                pl.semaphore_signal(
                    exit_sem, inc=1,
                    device_id=(nbr,), device_id_type=pltpu.DeviceIdType.MESH,
                )
            pl.semaphore_wait(exit_sem, 2)

    kg, vg = pl.pallas_call(
        body,
        out_shape=(
            jax.ShapeDtypeStruct((N_DEV, b, s, hd), K2.dtype),
            jax.ShapeDtypeStruct((N_DEV, b, s, hd), V2.dtype),
        ),
        in_specs=[
            pl.BlockSpec(memory_space=pltpu.MemorySpace.HBM),
            pl.BlockSpec(memory_space=pltpu.MemorySpace.HBM),
        ],
        out_specs=(
            pl.BlockSpec(memory_space=pltpu.MemorySpace.HBM),
            pl.BlockSpec(memory_space=pltpu.MemorySpace.HBM),
        ),
        scratch_shapes=[
            pltpu.SemaphoreType.DMA((2,)),
            pltpu.SemaphoreType.DMA((N_HOPS,)),
            pltpu.SemaphoreType.DMA((N_HOPS,)),
            pltpu.SemaphoreType.DMA((N_HOPS,)),
            pltpu.SemaphoreType.DMA((N_HOPS,)),
        ],
        compiler_params=pltpu.CompilerParams(collective_id=0),
    )(K2, V2)
    return kg, vg


def kernel(Q, K, V):
    b, s, h, d = Q.shape
    kg, vg = _allgather_kv(
        K.reshape(b, s, h * d), V.reshape(b, s, h * d)
    )
    Kf = kg.reshape(N_DEV, b, s, h, d).transpose(1, 0, 2, 3, 4).reshape(
        b, N_DEV * s, h, d
    )
    Vf = vg.reshape(N_DEV, b, s, h, d).transpose(1, 0, 2, 3, 4).reshape(
        b, N_DEV * s, h, d
    )

    scale = d ** -0.5
    S = jnp.einsum("bqhd,bkhd->bhqk", Q, Kf) * scale
    m = S.max(-1, keepdims=True)
    P = jnp.exp(S - m)
    P = P / P.sum(-1, keepdims=True)
    return jnp.einsum("bhqk,bkhd->bqhd", P, Vf).astype(Q.dtype)
